# baseline (speedup 1.0000x reference)
import jax
import jax.numpy as jnp
from jax import lax
from jax.experimental import pallas as pl
from jax.experimental.pallas import tpu as pltpu
from jax.experimental.pallas import tpu_sc as plsc

BATCH = 4096
VOCAB = 100000
SEQ = 50
DIM = 64
NC = 2
NS = 16
NW = NC * NS
BPW = BATCH // NW
PPW = BPW // 2
NBUF = 4
LANES = 16
CHUNKS = DIM // LANES
SEQ2 = 2 * SEQ
SEQ2P = 104


def _sc_body(idx_hbm, table_hbm, out_hbm, idx_v, rows_v, out_v, *sems):
    wid = lax.axis_index("s") * NC + lax.axis_index("c")
    pltpu.sync_copy(idx_hbm.at[pl.ds(wid * (PPW * SEQ2P), PPW * SEQ2P)], idx_v)

    def issue(p, b):
        pltpu.async_copy(table_hbm.at[idx_v.at[pl.ds(p * SEQ2P, SEQ2)]],
                         rows_v.at[b], sems[b])

    def consume(p, b):
        pltpu.make_async_copy(table_hbm.at[idx_v.at[pl.ds(p * SEQ2P, SEQ2)]],
                              rows_v.at[b], sems[b]).wait()
        rb = rows_v.at[b]
        for h in range(2):
            k0 = h * SEQ
            for c in range(CHUNKS):
                col = pl.ds(c * LANES, LANES)
                s0 = rb[k0, col]
                s1 = rb[k0 + 1, col]
                for k in range(k0 + 2, k0 + SEQ, 2):
                    s0 += rb[k, col]
                    s1 += rb[k + 1, col]
                out_v[pl.ds((p * 2 + h) * DIM + c * LANES, LANES)] = (
                    (s0 + s1) * (1.0 / SEQ))

    for b in range(NBUF):
        issue(b, b)

    groups = PPW // NBUF

    def group(g, issue_next):
        for b in range(NBUF):
            p = g * NBUF + b
            consume(p, b)
            if issue_next:
                issue(p + NBUF, b)

    def steady(g, carry):
        group(g, True)
        return carry

    lax.fori_loop(0, groups - 1, steady, 0)
    group(groups - 1, False)

    pltpu.sync_copy(out_v, out_hbm.at[pl.ds(wid * (BPW * DIM), BPW * DIM)])


_run = pl.kernel(
    _sc_body,
    out_type=jax.ShapeDtypeStruct((BATCH * DIM,), jnp.float32),
    mesh=plsc.VectorSubcoreMesh(core_axis_name="c", subcore_axis_name="s",
                                num_cores=NC, num_subcores=NS),
    scratch_types=[
        pltpu.VMEM((PPW * SEQ2P,), jnp.int32),
        pltpu.VMEM((NBUF, SEQ2, DIM), jnp.float32),
        pltpu.VMEM((BPW * DIM,), jnp.float32),
    ] + [pltpu.SemaphoreType.DMA] * NBUF,
    compiler_params=pltpu.CompilerParams(use_tc_tiling_on_sc=False),
)


def kernel(input_seq_batch, table):
    idx = jnp.pad(
        input_seq_batch.astype(jnp.int32).reshape(BATCH // 2, SEQ2),
        ((0, 0), (0, SEQ2P - SEQ2))).reshape((BATCH // 2) * SEQ2P)
    return _run(idx, table).reshape(BATCH, DIM)
